# baseline (speedup 1.0000x reference)
import jax
import jax.numpy as jnp
from jax import lax
from jax.experimental import pallas as pl
from jax.experimental.pallas import tpu as pltpu

NUM_MED = 2000
N_DIAGS = 128
N_PROCS = 64
K = 8
SD = N_DIAGS // K
SP = N_PROCS // K
STEPS = SD + SP
NEG = float(jnp.finfo(jnp.float32).min)


def _tree_max(xs):
    while len(xs) > 1:
        nxt = [jnp.maximum(xs[i], xs[i + 1]) for i in range(0, len(xs) - 1, 2)]
        if len(xs) % 2:
            nxt.append(xs[-1])
        xs = nxt
    return xs[0]


def _body(diags_ref, procs_ref, hl_ref, ll_ref, wm_ref, wp_ref, *refs):
    dblocks = refs[:K]
    pblocks = refs[K:2 * K]
    pre_ref = refs[2 * K]
    out_ref = refs[2 * K + 1]
    accd_ref = refs[2 * K + 2]
    accp_ref = refs[2 * K + 3]
    i = pl.program_id(0)
    iota = lax.broadcasted_iota(jnp.int32, (1, 128), 1)

    def masked(blocks, idx_ref, n, base):
        xs = []
        for j in range(K):
            lane = idx_ref[jnp.minimum(base + j, n - 1)] % 128
            m = jnp.where(iota == lane, 0.0, NEG)
            xs.append(blocks[j][...] + m)
        return _tree_max(xs)

    @pl.when(i == 0)
    def _():
        accd_ref[...] = masked(dblocks, diags_ref, N_DIAGS, K * i)

    @pl.when(jnp.logical_and(i > 0, i < SD))
    def _():
        accd_ref[...] = jnp.maximum(
            accd_ref[...], masked(dblocks, diags_ref, N_DIAGS, K * i))

    @pl.when(i == SD)
    def _():
        accp_ref[...] = masked(pblocks, procs_ref, N_PROCS, K * (i - SD))

    @pl.when(i > SD)
    def _():
        accp_ref[...] = jnp.maximum(
            accp_ref[...], masked(pblocks, procs_ref, N_PROCS, K * (i - SD)))

    @pl.when(i == STEPS - 1)
    def _():
        maxd = jnp.maximum(jnp.max(accd_ref[...], axis=1, keepdims=True), 0.0)
        maxp = jnp.maximum(jnp.max(accp_ref[...], axis=1, keepdims=True), 0.0)
        minus = jnp.logical_and(maxd < ll_ref[0], maxp < ll_ref[1])
        plus = jnp.logical_and(
            jnp.logical_not(minus),
            jnp.logical_or(maxd > hl_ref[0], maxp > hl_ref[1]),
        )
        delta = wp_ref[0] * plus.astype(jnp.float32) \
            - wm_ref[0] * minus.astype(jnp.float32)
        out_ref[...] = pre_ref[...] + jnp.transpose(delta)


def _dspec(j):
    return pl.BlockSpec(
        (NUM_MED, 128),
        lambda i, d, p, hl, ll, wm, wp, j=j: (
            0, d[jnp.minimum(K * i + j, N_DIAGS - 1)] // 128),
    )


def _pspec(j):
    return pl.BlockSpec(
        (NUM_MED, 128),
        lambda i, d, p, hl, ll, wm, wp, j=j: (
            0, p[jnp.clip(K * (i - SD) + j, 0, N_PROCS - 1)] // 128),
    )


def kernel(pre_prob, diag_med_effect, proc_med_effect, c1_high_limit,
           c1_low_limit, c1_minus_weight, c1_plus_weight, diags, procs):
    grid_spec = pltpu.PrefetchScalarGridSpec(
        num_scalar_prefetch=6,
        grid=(STEPS,),
        in_specs=[
            *[_dspec(j) for j in range(K)],
            *[_pspec(j) for j in range(K)],
            pl.BlockSpec((1, NUM_MED), lambda i, d, p, hl, ll, wm, wp: (0, 0)),
        ],
        out_specs=pl.BlockSpec(
            (1, NUM_MED), lambda i, d, p, hl, ll, wm, wp: (0, 0)),
        scratch_shapes=[
            pltpu.VMEM((NUM_MED, 128), jnp.float32),
            pltpu.VMEM((NUM_MED, 128), jnp.float32),
        ],
    )
    return pl.pallas_call(
        _body,
        grid_spec=grid_spec,
        out_shape=jax.ShapeDtypeStruct((1, NUM_MED), jnp.float32),
    )(diags.astype(jnp.int32), procs.astype(jnp.int32),
      c1_high_limit, c1_low_limit,
      jnp.reshape(c1_minus_weight, (1,)), jnp.reshape(c1_plus_weight, (1,)),
      *([diag_med_effect.T] * K),
      *([proc_med_effect.T] * K),
      pre_prob)
